# baseline (speedup 1.0000x reference)
import functools

import jax
import jax.numpy as jnp
from jax import lax
from jax.experimental import pallas as pl
from jax.experimental.pallas import tpu as pltpu
from jax.experimental.pallas import tpu_sc as plsc

N_ITEMS = 1000000
LATENT_DIM = 32
MARGIN = 0.5
N_NEG = 20
BATCH = 16384

NC = 2
NS = 16
L = 16
NW = NC * NS
TB = BATCH // NW
C = 64
NCHUNK = TB // C
KC = C * N_NEG
KR = KC // 128
NHALF = N_NEG // 2


def _rsqrt(x):
    i = plsc.bitcast(x, jnp.int32)
    i = jnp.int32(0x5F3759DF) - lax.shift_right_logical(i, 1)
    y = plsc.bitcast(i, jnp.float32)
    for _ in range(3):
        t = (0.5 * x) * y
        t = t * y
        y = y * (1.5 - t)
    return y


def _clip_scale(sq):
    return jnp.minimum(1.0, _rsqrt(jnp.maximum(sq, 1e-14)))


def _sc_body(i_hbm, j_hbm, kf_hbm, ut_hbm, it_hbm, wt_hbm, out_hbm,
             iidx_v, jidx_v, kidx_v, urows_v, irows_v, nrows_v, wtab_v,
             loss_v, sem):
    cid = lax.axis_index("c")
    sid = lax.axis_index("s")
    wid = sid * NC + cid
    base = wid * TB

    pltpu.sync_copy(wt_hbm, wtab_v)

    iota = lax.iota(jnp.int32, L)
    zerof = jnp.zeros((L,), jnp.float32)
    zeroi = jnp.zeros((L,), jnp.int32)

    def group_body(g, acc):
        rowv = g * L + iota

        @pl.loop(0, LATENT_DIM, init_carry=(zeroi, zerof, zerof, zerof))
        def pass1(d, carry):
            colv, u2, i2, ui = carry
            u = plsc.load_gather(urows_v, [rowv, colv])
            it = plsc.load_gather(irows_v, [rowv, colv])
            return colv + 1, u2 + u * u, i2 + it * it, ui + u * it

        _, u2, i2, ui = pass1
        su = _clip_scale(u2)
        si = _clip_scale(i2)
        a = MARGIN + si * si * i2 - 2.0 * su * si * ui
        su2 = 2.0 * su

        cnt = zeroi
        pr = zerof
        for h in range(2):
            nrow0 = rowv * N_NEG + h * NHALF
            init = (zeroi,) + (zerof,) * (2 * NHALF)

            @pl.loop(0, LATENT_DIM, init_carry=init)
            def neg_pass(d, carry):
                colv = carry[0]
                n2s = list(carry[1:1 + NHALF])
                uns = list(carry[1 + NHALF:])
                u = plsc.load_gather(urows_v, [rowv, colv])
                for n in range(NHALF):
                    x = plsc.load_gather(nrows_v, [nrow0 + n, colv])
                    n2s[n] = n2s[n] + x * x
                    uns[n] = uns[n] + u * x
                return (colv + 1,) + tuple(n2s) + tuple(uns)

            n2s = neg_pass[1:1 + NHALF]
            uns = neg_pass[1 + NHALF:]
            for n in range(NHALF):
                sn = _clip_scale(n2s[n])
                m = a - sn * (sn * n2s[n] - su2 * uns[n])
                pos = m > 0.0
                cnt = cnt + jnp.where(pos, 1, 0).astype(jnp.int32)
                pr = pr + jnp.where(pos, m, 0.0)

        w = plsc.load_gather(wtab_v, [cnt])
        return acc + w * pr

    @pl.loop(0, NCHUNK, init_carry=zerof)
    def chunk_loop(c, loss_acc):
        cb = base + c * C
        pltpu.sync_copy(i_hbm.at[pl.ds(cb, C)], iidx_v)
        pltpu.sync_copy(j_hbm.at[pl.ds(cb, C)], jidx_v)
        pltpu.sync_copy(kf_hbm.at[pl.ds(cb * N_NEG, KC)], kidx_v)
        cps = [pltpu.async_copy(ut_hbm.at[iidx_v], urows_v, sem),
               pltpu.async_copy(it_hbm.at[jidx_v], irows_v, sem)]
        for r in range(KR):
            cps.append(pltpu.async_copy(it_hbm.at[kidx_v.at[pl.ds(r * 128, 128)]],
                                        nrows_v.at[pl.ds(r * 128, 128)],
                                        sem))
        for cp in cps:
            cp.wait()

        acc = loss_acc
        for g in range(C // L):
            acc = group_body(g, acc)
        return acc

    loss_v[...] = chunk_loop
    pltpu.sync_copy(loss_v, out_hbm.at[pl.ds(wid * L, L)])


@functools.cache
def _sc_kernel():
    return functools.partial(
        pl.kernel,
        out_type=jax.ShapeDtypeStruct((NW * L,), jnp.float32),
        mesh=plsc.VectorSubcoreMesh(core_axis_name="c", subcore_axis_name="s",
                                    num_cores=NC, num_subcores=NS),
        scratch_types=[
            pltpu.VMEM((C,), jnp.int32),
            pltpu.VMEM((C,), jnp.int32),
            pltpu.VMEM((KC,), jnp.int32),
            pltpu.VMEM((C, LATENT_DIM), jnp.float32),
            pltpu.VMEM((C, LATENT_DIM), jnp.float32),
            pltpu.VMEM((KC, LATENT_DIM), jnp.float32),
            pltpu.VMEM((32,), jnp.float32),
            pltpu.VMEM((L,), jnp.float32),
            pltpu.SemaphoreType.DMA,
        ],
        compiler_params=pltpu.CompilerParams(needs_layout_passes=False,
                                             use_tc_tiling_on_sc=False),
    )(_sc_body)


@jax.jit
def kernel(i, j, k, user_table, item_table):
    i = i.astype(jnp.int32)
    j = j.astype(jnp.int32)
    kf = k.astype(jnp.int32).reshape(BATCH * N_NEG)
    cnts = jnp.arange(N_NEG + 1, dtype=jnp.float32)
    wtab = jnp.log(N_ITEMS * cnts / N_NEG + 1.0)
    wtab = jnp.concatenate(
        [wtab, jnp.zeros((32 - (N_NEG + 1),), jnp.float32)])
    partial = _sc_kernel()(i, j, kf, user_table, item_table, wtab)
    return jnp.sum(partial)


# speedup vs baseline: 1.2278x; 1.2278x over previous
import functools

import jax
import jax.numpy as jnp
from jax import lax
from jax.experimental import pallas as pl
from jax.experimental.pallas import tpu as pltpu
from jax.experimental.pallas import tpu_sc as plsc

N_ITEMS = 1000000
LATENT_DIM = 32
MARGIN = 0.5
N_NEG = 20
BATCH = 16384

NC = 2
NS = 16
L = 16
NW = NC * NS
TB = BATCH // NW
C = 64
NCHUNK = TB // C
KC = C * N_NEG
KR = KC // 128
NHALF = N_NEG // 2


def _rsqrt(x):
    i = plsc.bitcast(x, jnp.int32)
    i = jnp.int32(0x5F3759DF) - lax.shift_right_logical(i, 1)
    y = plsc.bitcast(i, jnp.float32)
    for _ in range(3):
        t = (0.5 * x) * y
        t = t * y
        y = y * (1.5 - t)
    return y


def _clip_scale(sq):
    return jnp.minimum(1.0, _rsqrt(jnp.maximum(sq, 1e-14)))


def _sc_body(i_hbm, j_hbm, kf_hbm, ut_hbm, it_hbm, wt_hbm, out_hbm,
             iidx0, jidx0, kidx0, urows0, irows0, nrows0,
             iidx1, jidx1, kidx1, urows1, irows1, nrows1,
             wtab_v, loss_v, sem0, sem1):
    cid = lax.axis_index("c")
    sid = lax.axis_index("s")
    wid = sid * NC + cid
    base = wid * TB

    pltpu.sync_copy(wt_hbm, wtab_v)

    iota = lax.iota(jnp.int32, L)
    zerof = jnp.zeros((L,), jnp.float32)
    zeroi = jnp.zeros((L,), jnp.int32)

    bufs = ((iidx0, jidx0, kidx0, urows0, irows0, nrows0, sem0),
            (iidx1, jidx1, kidx1, urows1, irows1, nrows1, sem1))

    def fire(c, p):
        iidx, jidx, kidx, urows, irows, nrows, sem = bufs[p]
        cb = base + c * C
        pltpu.sync_copy(i_hbm.at[pl.ds(cb, C)], iidx)
        pltpu.sync_copy(j_hbm.at[pl.ds(cb, C)], jidx)
        pltpu.sync_copy(kf_hbm.at[pl.ds(cb * N_NEG, KC)], kidx)
        pltpu.async_copy(ut_hbm.at[iidx], urows, sem)
        pltpu.async_copy(it_hbm.at[jidx], irows, sem)
        for r in range(KR):
            pltpu.async_copy(it_hbm.at[kidx.at[pl.ds(r * 128, 128)]],
                             nrows.at[pl.ds(r * 128, 128)], sem)

    def drain(p):
        iidx, jidx, kidx, urows, irows, nrows, sem = bufs[p]
        pltpu.make_async_copy(ut_hbm.at[iidx], urows, sem).wait()
        pltpu.make_async_copy(it_hbm.at[jidx], irows, sem).wait()
        for r in range(KR):
            pltpu.make_async_copy(it_hbm.at[kidx.at[pl.ds(r * 128, 128)]],
                                  nrows.at[pl.ds(r * 128, 128)], sem).wait()

    def group_body(g, acc, p):
        urows, irows, nrows = bufs[p][3], bufs[p][4], bufs[p][5]
        rowv = g * L + iota

        @pl.loop(0, LATENT_DIM, init_carry=(iota, zerof, zerof, zerof))
        def pass1(d, carry):
            colv, u2, i2, ui = carry
            u = plsc.load_gather(urows, [rowv, colv])
            it = plsc.load_gather(irows, [rowv, colv])
            return ((colv + 1) & (LATENT_DIM - 1),
                    u2 + u * u, i2 + it * it, ui + u * it)

        _, u2, i2, ui = pass1
        su = _clip_scale(u2)
        si = _clip_scale(i2)
        a = MARGIN + si * si * i2 - 2.0 * su * si * ui
        su2 = 2.0 * su

        cnt = zeroi
        pr = zerof
        for h in range(2):
            nrow0 = rowv * N_NEG + h * NHALF
            init = (iota,) + (zerof,) * (2 * NHALF)

            @pl.loop(0, LATENT_DIM, init_carry=init)
            def neg_pass(d, carry):
                colv = carry[0]
                n2s = list(carry[1:1 + NHALF])
                uns = list(carry[1 + NHALF:])
                u = plsc.load_gather(urows, [rowv, colv])
                for n in range(NHALF):
                    x = plsc.load_gather(nrows, [nrow0 + n, colv])
                    n2s[n] = n2s[n] + x * x
                    uns[n] = uns[n] + u * x
                return ((colv + 1) & (LATENT_DIM - 1),) + tuple(n2s) + tuple(uns)

            n2s = neg_pass[1:1 + NHALF]
            uns = neg_pass[1 + NHALF:]
            for n in range(NHALF):
                sn = _clip_scale(n2s[n])
                m = a - sn * (sn * n2s[n] - su2 * uns[n])
                pos = m > 0.0
                cnt = cnt + jnp.where(pos, 1, 0).astype(jnp.int32)
                pr = pr + jnp.where(pos, m, 0.0)

        w = plsc.load_gather(wtab_v, [cnt])
        return acc + w * pr

    def compute(acc, p):
        for g in range(C // L):
            acc = group_body(g, acc, p)
        return acc

    fire(0, 0)

    @pl.loop(0, NCHUNK, step=2, init_carry=zerof)
    def chunk_loop(c, acc):
        fire(c + 1, 1)
        drain(0)
        acc = compute(acc, 0)

        @pl.when(c + 2 < NCHUNK)
        def _():
            fire(c + 2, 0)

        drain(1)
        return compute(acc, 1)

    loss_v[...] = chunk_loop
    pltpu.sync_copy(loss_v, out_hbm.at[pl.ds(wid * L, L)])


@functools.cache
def _sc_kernel():
    idx_scratch = [pltpu.VMEM((C,), jnp.int32),
                   pltpu.VMEM((C,), jnp.int32),
                   pltpu.VMEM((KC,), jnp.int32),
                   pltpu.VMEM((C, LATENT_DIM), jnp.float32),
                   pltpu.VMEM((C, LATENT_DIM), jnp.float32),
                   pltpu.VMEM((KC, LATENT_DIM), jnp.float32)]
    return functools.partial(
        pl.kernel,
        out_type=jax.ShapeDtypeStruct((NW * L,), jnp.float32),
        mesh=plsc.VectorSubcoreMesh(core_axis_name="c", subcore_axis_name="s",
                                    num_cores=NC, num_subcores=NS),
        scratch_types=idx_scratch + idx_scratch + [
            pltpu.VMEM((32,), jnp.float32),
            pltpu.VMEM((L,), jnp.float32),
            pltpu.SemaphoreType.DMA,
            pltpu.SemaphoreType.DMA,
        ],
        compiler_params=pltpu.CompilerParams(needs_layout_passes=False,
                                             use_tc_tiling_on_sc=False),
    )(_sc_body)


@jax.jit
def kernel(i, j, k, user_table, item_table):
    i = i.astype(jnp.int32)
    j = j.astype(jnp.int32)
    kf = k.astype(jnp.int32).reshape(BATCH * N_NEG)
    cnts = jnp.arange(N_NEG + 1, dtype=jnp.float32)
    wtab = jnp.log(N_ITEMS * cnts / N_NEG + 1.0)
    wtab = jnp.concatenate(
        [wtab, jnp.zeros((32 - (N_NEG + 1),), jnp.float32)])
    partial = _sc_kernel()(i, j, kf, user_table, item_table, wtab)
    return jnp.sum(partial)


# speedup vs baseline: 1.2392x; 1.0093x over previous
import functools

import jax
import jax.numpy as jnp
from jax import lax
from jax.experimental import pallas as pl
from jax.experimental.pallas import tpu as pltpu
from jax.experimental.pallas import tpu_sc as plsc

N_ITEMS = 1000000
LATENT_DIM = 32
MARGIN = 0.5
N_NEG = 20
BATCH = 16384

NC = 2
NS = 16
L = 16
NW = NC * NS
TB = BATCH // NW
C = 64
NCHUNK = TB // C
KC = C * N_NEG
KR = KC // 128
NHALF = N_NEG // 2


def _rsqrt(x):
    i = plsc.bitcast(x, jnp.int32)
    i = jnp.int32(0x5F3759DF) - lax.shift_right_logical(i, 1)
    y = plsc.bitcast(i, jnp.float32)
    for _ in range(3):
        t = (0.5 * x) * y
        t = t * y
        y = y * (1.5 - t)
    return y


def _clip_scale(sq):
    return jnp.minimum(1.0, _rsqrt(jnp.maximum(sq, 1e-14)))


def _sc_body(i_hbm, j_hbm, kf_hbm, ut_hbm, it_hbm, wt_hbm, out_hbm,
             iidx_v, jidx_v, kidx_v,
             urows0, irows0, nrows0,
             urows1, irows1, nrows1,
             wtab_v, loss_v, sem0, sem1):
    cid = lax.axis_index("c")
    sid = lax.axis_index("s")
    wid = sid * NC + cid
    base = wid * TB

    pltpu.sync_copy(wt_hbm, wtab_v)
    pltpu.sync_copy(i_hbm.at[pl.ds(base, TB)], iidx_v)
    pltpu.sync_copy(j_hbm.at[pl.ds(base, TB)], jidx_v)
    pltpu.sync_copy(kf_hbm.at[pl.ds(base * N_NEG, TB * N_NEG)], kidx_v)

    iota = lax.iota(jnp.int32, L)
    zerof = jnp.zeros((L,), jnp.float32)
    zeroi = jnp.zeros((L,), jnp.int32)

    bufs = ((urows0, irows0, nrows0, sem0),
            (urows1, irows1, nrows1, sem1))

    def fire(c, p):
        urows, irows, nrows, sem = bufs[p]
        pltpu.async_copy(ut_hbm.at[iidx_v.at[pl.ds(c * C, C)]], urows, sem)
        pltpu.async_copy(it_hbm.at[jidx_v.at[pl.ds(c * C, C)]], irows, sem)
        pltpu.async_copy(it_hbm.at[kidx_v.at[pl.ds(c * KC, KC)]], nrows, sem)

    def drain(c, p):
        urows, irows, nrows, sem = bufs[p]
        pltpu.make_async_copy(ut_hbm.at[iidx_v.at[pl.ds(c * C, C)]],
                              urows, sem).wait()
        pltpu.make_async_copy(it_hbm.at[jidx_v.at[pl.ds(c * C, C)]],
                              irows, sem).wait()
        pltpu.make_async_copy(it_hbm.at[kidx_v.at[pl.ds(c * KC, KC)]],
                              nrows, sem).wait()

    def group_body(g, acc, p):
        urows, irows, nrows = bufs[p][0], bufs[p][1], bufs[p][2]
        rowv = g * L + iota

        @pl.loop(0, LATENT_DIM, init_carry=(iota, zerof, zerof, zerof))
        def pass1(d, carry):
            colv, u2, i2, ui = carry
            u = plsc.load_gather(urows, [rowv, colv])
            it = plsc.load_gather(irows, [rowv, colv])
            return ((colv + 1) & (LATENT_DIM - 1),
                    u2 + u * u, i2 + it * it, ui + u * it)

        _, u2, i2, ui = pass1
        su = _clip_scale(u2)
        si = _clip_scale(i2)
        a = MARGIN + si * si * i2 - 2.0 * su * si * ui
        su2 = 2.0 * su

        cnt = zeroi
        pr = zerof
        for h in range(2):
            nrow0 = rowv * N_NEG + h * NHALF
            init = (iota,) + (zerof,) * (2 * NHALF)

            @pl.loop(0, LATENT_DIM, init_carry=init)
            def neg_pass(d, carry):
                colv = carry[0]
                n2s = list(carry[1:1 + NHALF])
                uns = list(carry[1 + NHALF:])
                u = plsc.load_gather(urows, [rowv, colv])
                for n in range(NHALF):
                    x = plsc.load_gather(nrows, [nrow0 + n, colv])
                    n2s[n] = n2s[n] + x * x
                    uns[n] = uns[n] + u * x
                return ((colv + 1) & (LATENT_DIM - 1),) + tuple(n2s) + tuple(uns)

            n2s = neg_pass[1:1 + NHALF]
            uns = neg_pass[1 + NHALF:]
            for n in range(NHALF):
                sn = _clip_scale(n2s[n])
                m = a - sn * (sn * n2s[n] - su2 * uns[n])
                pos = m > 0.0
                cnt = cnt + jnp.where(pos, 1, 0).astype(jnp.int32)
                pr = pr + jnp.where(pos, m, 0.0)

        w = plsc.load_gather(wtab_v, [cnt])
        return acc + w * pr

    def compute(acc, p):
        for g in range(C // L):
            acc = group_body(g, acc, p)
        return acc

    fire(0, 0)

    @pl.loop(0, NCHUNK, step=2, init_carry=zerof)
    def chunk_loop(c, acc):
        fire(c + 1, 1)
        drain(c, 0)
        acc = compute(acc, 0)

        @pl.when(c + 2 < NCHUNK)
        def _():
            fire(c + 2, 0)

        drain(c + 1, 1)
        return compute(acc, 1)

    loss_v[...] = chunk_loop
    pltpu.sync_copy(loss_v, out_hbm.at[pl.ds(wid * L, L)])


@functools.cache
def _sc_kernel():
    row_scratch = [pltpu.VMEM((C, LATENT_DIM), jnp.float32),
                   pltpu.VMEM((C, LATENT_DIM), jnp.float32),
                   pltpu.VMEM((KC, LATENT_DIM), jnp.float32)]
    return functools.partial(
        pl.kernel,
        out_type=jax.ShapeDtypeStruct((NW * L,), jnp.float32),
        mesh=plsc.VectorSubcoreMesh(core_axis_name="c", subcore_axis_name="s",
                                    num_cores=NC, num_subcores=NS),
        scratch_types=[
            pltpu.VMEM((TB,), jnp.int32),
            pltpu.VMEM((TB,), jnp.int32),
            pltpu.VMEM((TB * N_NEG,), jnp.int32),
        ] + row_scratch + row_scratch + [
            pltpu.VMEM((32,), jnp.float32),
            pltpu.VMEM((L,), jnp.float32),
            pltpu.SemaphoreType.DMA,
            pltpu.SemaphoreType.DMA,
        ],
        compiler_params=pltpu.CompilerParams(needs_layout_passes=False,
                                             use_tc_tiling_on_sc=False),
    )(_sc_body)


@jax.jit
def kernel(i, j, k, user_table, item_table):
    i = i.astype(jnp.int32)
    j = j.astype(jnp.int32)
    kf = k.astype(jnp.int32).reshape(BATCH * N_NEG)
    cnts = jnp.arange(N_NEG + 1, dtype=jnp.float32)
    wtab = jnp.log(N_ITEMS * cnts / N_NEG + 1.0)
    wtab = jnp.concatenate(
        [wtab, jnp.zeros((32 - (N_NEG + 1),), jnp.float32)])
    partial = _sc_kernel()(i, j, kf, user_table, item_table, wtab)
    return jnp.sum(partial)
